# baseline (speedup 1.0000x reference)
import functools

import jax
import jax.numpy as jnp
from jax import lax
from jax.experimental import pallas as pl
from jax.experimental.pallas import tpu as pltpu
from jax.experimental.pallas import tpu_sc as plsc

_NC, _NS, _L = 2, 16, 16
_NW = _NC * _NS

_V = 50000
_N = 4096
_B = 32
_LC = 32
_LA = 24
_D = 256
_NITEMS = _N + _B
_NPAD = 4608
_PER_W = _NPAD // _NW
_CH = 4
_NCHUNK = _PER_W // _CH
_NBUF = 4
_DP = _D // 2


def _pool_sc_kernel(embp_h, emb_h, idx_h, coef_h, a_h, out_h, aout_h,
                    idx_v, rows_v, coef_v, acc_v, aidx_v, arows_v,
                    sems, asem):
    w = lax.axis_index("s") * _NC + lax.axis_index("c")
    base = w * _PER_W

    pltpu.sync_copy(idx_h.at[pl.ds(base * _LC, _PER_W * _LC)], idx_v)
    pltpu.sync_copy(coef_h.at[pl.ds(base, _PER_W)], coef_v)
    pltpu.sync_copy(a_h.at[pl.ds(w * _LA, _LA)], aidx_v)
    pltpu.async_copy(emb_h.at[aidx_v], arows_v, asem)

    def _gather(ci, buf):
        return pltpu.make_async_copy(
            embp_h.at[idx_v.at[pl.ds(ci * _CH * _LC, _CH * _LC)]],
            rows_v.at[buf], sems.at[buf])

    def _fire(ci, buf):
        pltpu.async_copy(
            embp_h.at[idx_v.at[pl.ds(ci * _CH * _LC, _CH * _LC)]],
            rows_v.at[buf], sems.at[buf])

    for b0 in range(_NBUF):
        _fire(b0, b0)

    def outer(it, carry):
        for buf in range(_NBUF):
            ci = it * _NBUF + buf
            _gather(ci, buf).wait()

            def item_body(j, carry2):
                coefj = coef_v[ci * _CH + j]
                scale = 1.0 - coefj
                r0 = j * _LC

                def _row(l, sl):
                    x = rows_v[buf, r0 + l, sl]
                    lo = lax.bitcast_convert_type(x << 16, jnp.float32)
                    hi = lax.bitcast_convert_type(x & jnp.int32(-65536),
                                                  jnp.float32)
                    return lo, hi

                for g in range(_DP // _L):
                    sl = pl.ds(g * _L, _L)
                    e0, o0 = _row(0, sl)
                    ea, oa = [e0 * scale], [o0 * scale]
                    for k in range(1, 4):
                        ek, ok = _row(k, sl)
                        ea.append(ek)
                        oa.append(ok)
                    for lb in range(4, _LC, 4):
                        for k in range(4):
                            ek, ok = _row(lb + k, sl)
                            ea[k] = ea[k] + ek
                            oa[k] = oa[k] + ok
                    acc_e = (ea[0] + ea[1]) + (ea[2] + ea[3])
                    acc_o = (oa[0] + oa[1]) + (oa[2] + oa[3])
                    acc_v[buf * _CH + j, pl.ds(g * _L, _L)] = acc_e
                    acc_v[buf * _CH + j, pl.ds(_DP + g * _L, _L)] = acc_o
                return carry2

            lax.fori_loop(0, _CH, item_body, 0)

            @pl.when(ci + _NBUF < _NCHUNK)
            def _():
                _fire(ci + _NBUF, buf)

        pltpu.sync_copy(acc_v, out_h.at[pl.ds(base + it * _NBUF * _CH,
                                              _NBUF * _CH)])
        return carry

    lax.fori_loop(0, _NCHUNK // _NBUF, outer, 0)

    pltpu.make_async_copy(emb_h.at[aidx_v], arows_v, asem).wait()
    pltpu.sync_copy(arows_v, aout_h.at[w])


def _gather_top_sc_kernel(emb_h, ctok_h, best_h, cout_h,
                          bidx_v, ctoksel_v, cemb_v, sem):
    w = lax.axis_index("s") * _NC + lax.axis_index("c")
    pltpu.sync_copy(best_h, bidx_v)
    pltpu.async_copy(ctok_h.at[bidx_v], ctoksel_v, sem).wait()
    pltpu.async_copy(emb_h.at[ctoksel_v.at[w, pl.ds(0, _LC)]], cemb_v,
                     sem).wait()
    pltpu.sync_copy(cemb_v, cout_h.at[w])


def _score_tc_kernel(s_ref, best_ref):
    S = s_ref[...]
    cs = S[:_N, :]
    qs = S[_N:_N + _B, :]
    cn = cs * lax.rsqrt(jnp.sum(cs * cs, axis=1, keepdims=True) + 1e-30)
    scores = lax.dot_general(qs, cn, (((1,), (1,)), ((), ())),
                             preferred_element_type=jnp.float32)
    m = jnp.max(scores, axis=1, keepdims=True)
    ii = lax.broadcasted_iota(jnp.int32, scores.shape, 1)
    cand = jnp.where(scores >= m, ii, jnp.int32(2 ** 30))
    best_ref[...] = jnp.min(cand, axis=1)


def _loss_tc_kernel(alen_ref, a_ref, c_ref, out_ref):
    b = pl.program_id(0)
    A = a_ref[0]
    C = c_ref[0]
    an = A * lax.rsqrt(jnp.sum(A * A, axis=1, keepdims=True))
    cn = C * lax.rsqrt(jnp.sum(C * C, axis=1, keepdims=True))
    cos = lax.dot_general(an, cn, (((1,), (1,)), ((), ())),
                          preferred_element_type=jnp.float32)
    em = jnp.exp(-0.5 * (cos - 1.0) ** 2 / (0.001 ** 2))
    sm = em / (jnp.sum(em, axis=1, keepdims=True) + 1e-10)
    mm = jnp.sum(em * sm, axis=1, keepdims=True)
    al = alen_ref[b].astype(jnp.float32)
    mask = (lax.broadcasted_iota(jnp.int32, (_LA, 1), 0)
            < alen_ref[b]).astype(jnp.float32)
    tot = jnp.sum(mm * mask)
    loss_b = 1.0 - tot / al

    @pl.when(b == 0)
    def _():
        out_ref[0, 0] = 0.0

    out_ref[0, 0] += loss_b / _B


def _sc_mesh():
    return plsc.VectorSubcoreMesh(core_axis_name="c", subcore_axis_name="s",
                                  num_cores=_NC, num_subcores=_NS)


def _pool_call(*args):
    return pl.kernel(
        _pool_sc_kernel,
        out_type=(jax.ShapeDtypeStruct((_NPAD, _D), jnp.float32),
                  jax.ShapeDtypeStruct((_B, _LA, _D), jnp.float32)),
        mesh=_sc_mesh(),
        scratch_types=[
            pltpu.VMEM((_PER_W * _LC,), jnp.int32),
            pltpu.VMEM((_NBUF, _CH * _LC, _DP), jnp.int32),
            pltpu.VMEM((_PER_W, _L), jnp.float32),
            pltpu.VMEM((_NBUF * _CH, _D), jnp.float32),
            pltpu.VMEM((_LA,), jnp.int32),
            pltpu.VMEM((_LA, _D), jnp.float32),
            pltpu.SemaphoreType.DMA((_NBUF,)),
            pltpu.SemaphoreType.DMA,
        ],
    )(*args)


def _gather_top_call(*args):
    return pl.kernel(
        _gather_top_sc_kernel,
        out_type=jax.ShapeDtypeStruct((_B, _LC, _D), jnp.float32),
        mesh=_sc_mesh(),
        scratch_types=[
            pltpu.VMEM((_B,), jnp.int32),
            pltpu.VMEM((_B, 128), jnp.int32),
            pltpu.VMEM((_LC, _D), jnp.float32),
            pltpu.SemaphoreType.DMA,
        ],
    )(*args)


def kernel(emb, q, c, a, qlen, clen, alen):
    emb = emb.astype(jnp.float32)
    ctok = c[:, :, 0].astype(jnp.int32)
    qtok = q[:, :, 0].astype(jnp.int32)
    clen = clen.astype(jnp.int32)
    qlen = qlen.astype(jnp.int32)
    alen = alen.astype(jnp.int32)
    a32 = a.astype(jnp.int32)

    pos = jnp.arange(_LC, dtype=jnp.int32)[None, :]
    cidx = jnp.where(pos < clen[:, None], ctok, ctok[:, :1])
    qidx = jnp.where(pos < qlen[:, None], qtok, qtok[:, :1])
    ccoef = (_LC - clen).astype(jnp.float32)
    qcoef = (_LC - qlen).astype(jnp.float32)

    idx_all = jnp.concatenate(
        [cidx, qidx, jnp.zeros((_NPAD - _NITEMS, _LC), jnp.int32)], axis=0)
    coef_all = jnp.concatenate(
        [ccoef, qcoef, jnp.zeros((_NPAD - _NITEMS,), jnp.float32)], axis=0)
    idx_flat = idx_all.reshape(-1)
    coef_b = coef_all[:, None] + jnp.zeros((_NPAD, _L), jnp.float32)
    a_flat = a32.reshape(-1)

    bu = lax.bitcast_convert_type(emb, jnp.uint32)
    rb = (bu + jnp.uint32(0x7FFF) + ((bu >> 16) & jnp.uint32(1))) >> 16
    emb_pk = lax.bitcast_convert_type(
        rb[:, :_DP] | (rb[:, _DP:] << 16), jnp.int32)
    ssum, a_emb = _pool_call(emb_pk, emb, idx_flat, coef_b, a_flat)

    best = pl.pallas_call(
        _score_tc_kernel,
        out_shape=jax.ShapeDtypeStruct((_B,), jnp.int32),
    )(ssum)

    ctok_pad = jnp.pad(ctok, ((0, 0), (0, 128 - _LC)))
    c_emb = _gather_top_call(emb, ctok_pad, best)

    loss = pl.pallas_call(
        _loss_tc_kernel,
        grid=(_B,),
        in_specs=[
            pl.BlockSpec(memory_space=pltpu.SMEM),
            pl.BlockSpec((1, _LA, _D), lambda b: (b, 0, 0)),
            pl.BlockSpec((1, _LC, _D), lambda b: (b, 0, 0)),
        ],
        out_specs=pl.BlockSpec(memory_space=pltpu.SMEM),
        out_shape=jax.ShapeDtypeStruct((1, 1), jnp.float32),
    )(alen, a_emb, c_emb)

    return loss[0, 0]


# speedup vs baseline: 1.9040x; 1.9040x over previous
import functools

import jax
import jax.numpy as jnp
from jax import lax
from jax.experimental import pallas as pl
from jax.experimental.pallas import tpu as pltpu
from jax.experimental.pallas import tpu_sc as plsc

_NC, _NS, _L = 2, 16, 16
_NW = _NC * _NS

_V = 50000
_N = 4096
_B = 32
_LC = 32
_LA = 24
_D = 256
_NITEMS = _N + _B
_NPAD = 4352
_PER_W = _NPAD // _NW
_CH = 4
_NCHUNK = _PER_W // _CH
_NBUF = 2
_DP = _D // 2


def _pool_sc_kernel(embp_h, emb_h, idx_h, coef_h, a_h, out_h, aout_h,
                    idx_v, rows_v, coef_v, acc_v, aidx_v, arows_v,
                    sems, asem):
    w = lax.axis_index("s") * _NC + lax.axis_index("c")
    base = w * _PER_W

    pltpu.sync_copy(idx_h.at[pl.ds(base * _LC, _PER_W * _LC)], idx_v)
    pltpu.sync_copy(coef_h.at[pl.ds(base, _PER_W)], coef_v)
    pltpu.sync_copy(a_h.at[pl.ds(w * _LA, _LA)], aidx_v)
    pltpu.async_copy(emb_h.at[aidx_v], arows_v, asem)

    def _gather(ci, buf):
        return pltpu.make_async_copy(
            embp_h.at[idx_v.at[pl.ds(ci * _CH * _LC, _CH * _LC)]],
            rows_v.at[buf], sems.at[buf])

    def _fire(ci, buf):
        pltpu.async_copy(
            embp_h.at[idx_v.at[pl.ds(ci * _CH * _LC, _CH * _LC)]],
            rows_v.at[buf], sems.at[buf])

    for b0 in range(_NBUF):
        _fire(b0, b0)

    def outer(it, carry):
        for buf in range(_NBUF):
            ci = it * _NBUF + buf
            _gather(ci, buf).wait()

            def item_body(j, carry2):
                coefj = coef_v[ci * _CH + j]
                scale = 1.0 - coefj
                r0 = j * _LC

                def _row(l, sl):
                    x = rows_v[buf, r0 + l, sl]
                    lo = lax.bitcast_convert_type(x << 16, jnp.float32)
                    hi = lax.bitcast_convert_type(x & jnp.int32(-65536),
                                                  jnp.float32)
                    return lo, hi

                for g in range(_DP // _L):
                    sl = pl.ds(g * _L, _L)
                    e0, o0 = _row(0, sl)
                    ea, oa = [e0 * scale], [o0 * scale]
                    for k in range(1, 4):
                        ek, ok = _row(k, sl)
                        ea.append(ek)
                        oa.append(ok)
                    for lb in range(4, _LC, 4):
                        for k in range(4):
                            ek, ok = _row(lb + k, sl)
                            ea[k] = ea[k] + ek
                            oa[k] = oa[k] + ok
                    acc_e = (ea[0] + ea[1]) + (ea[2] + ea[3])
                    acc_o = (oa[0] + oa[1]) + (oa[2] + oa[3])
                    acc_v[buf * _CH + j, pl.ds(g * _L, _L)] = acc_e
                    acc_v[buf * _CH + j, pl.ds(_DP + g * _L, _L)] = acc_o
                return carry2

            lax.fori_loop(0, _CH, item_body, 0)

            @pl.when(ci + _NBUF < _NCHUNK)
            def _():
                _fire(ci + _NBUF, buf)

        pltpu.sync_copy(acc_v, out_h.at[pl.ds(base + it * _NBUF * _CH,
                                              _NBUF * _CH)])
        return carry

    lax.fori_loop(0, _NCHUNK // _NBUF, outer, 0)

    pltpu.make_async_copy(emb_h.at[aidx_v], arows_v, asem).wait()
    pltpu.sync_copy(arows_v, aout_h.at[w])


def _gather_top_sc_kernel(emb_h, ctok_h, best_h, cout_h,
                          bidx_v, ctoksel_v, cemb_v, sem):
    w = lax.axis_index("s") * _NC + lax.axis_index("c")
    pltpu.sync_copy(best_h, bidx_v)
    pltpu.async_copy(ctok_h.at[bidx_v], ctoksel_v, sem).wait()
    pltpu.async_copy(emb_h.at[ctoksel_v.at[w, pl.ds(0, _LC)]], cemb_v,
                     sem).wait()
    pltpu.sync_copy(cemb_v, cout_h.at[w])


def _pack_tc_kernel(e_ref, p_ref):
    bu = lax.bitcast_convert_type(e_ref[...], jnp.uint32)
    rb = (bu + jnp.uint32(0x7FFF) + ((bu >> 16) & jnp.uint32(1))) >> 16
    p_ref[...] = lax.bitcast_convert_type(
        rb[:, :_DP] | (rb[:, _DP:] << 16), jnp.int32)


def _score_tc_kernel(s_ref, best_ref):
    S = s_ref[...]
    cs = S[:_N, :]
    qs = S[_N:_N + _B, :]
    cn = cs * lax.rsqrt(jnp.sum(cs * cs, axis=1, keepdims=True) + 1e-30)
    scores = lax.dot_general(qs, cn, (((1,), (1,)), ((), ())),
                             preferred_element_type=jnp.float32)
    m = jnp.max(scores, axis=1, keepdims=True)
    ii = lax.broadcasted_iota(jnp.int32, scores.shape, 1)
    cand = jnp.where(scores >= m, ii, jnp.int32(2 ** 30))
    best_ref[...] = jnp.min(cand, axis=1)


def _loss_tc_kernel(alen_ref, a_ref, c_ref, out_ref):
    b = pl.program_id(0)
    A = a_ref[0]
    C = c_ref[0]
    an = A * lax.rsqrt(jnp.sum(A * A, axis=1, keepdims=True))
    cn = C * lax.rsqrt(jnp.sum(C * C, axis=1, keepdims=True))
    cos = lax.dot_general(an, cn, (((1,), (1,)), ((), ())),
                          preferred_element_type=jnp.float32)
    em = jnp.exp(-0.5 * (cos - 1.0) ** 2 / (0.001 ** 2))
    sm = em / (jnp.sum(em, axis=1, keepdims=True) + 1e-10)
    mm = jnp.sum(em * sm, axis=1, keepdims=True)
    al = alen_ref[b].astype(jnp.float32)
    mask = (lax.broadcasted_iota(jnp.int32, (_LA, 1), 0)
            < alen_ref[b]).astype(jnp.float32)
    tot = jnp.sum(mm * mask)
    loss_b = 1.0 - tot / al

    @pl.when(b == 0)
    def _():
        out_ref[0, 0] = 0.0

    out_ref[0, 0] += loss_b / _B


def _sc_mesh():
    return plsc.VectorSubcoreMesh(core_axis_name="c", subcore_axis_name="s",
                                  num_cores=_NC, num_subcores=_NS)


def _pool_call(*args):
    return pl.kernel(
        _pool_sc_kernel,
        out_type=(jax.ShapeDtypeStruct((_NPAD, _D), jnp.float32),
                  jax.ShapeDtypeStruct((_B, _LA, _D), jnp.float32)),
        mesh=_sc_mesh(),
        scratch_types=[
            pltpu.VMEM((_PER_W * _LC,), jnp.int32),
            pltpu.VMEM((_NBUF, _CH * _LC, _DP), jnp.int32),
            pltpu.VMEM((_PER_W, _L), jnp.float32),
            pltpu.VMEM((_NBUF * _CH, _D), jnp.float32),
            pltpu.VMEM((_LA,), jnp.int32),
            pltpu.VMEM((_LA, _D), jnp.float32),
            pltpu.SemaphoreType.DMA((_NBUF,)),
            pltpu.SemaphoreType.DMA,
        ],
    )(*args)


def _gather_top_call(*args):
    return pl.kernel(
        _gather_top_sc_kernel,
        out_type=jax.ShapeDtypeStruct((_B, _LC, _D), jnp.float32),
        mesh=_sc_mesh(),
        scratch_types=[
            pltpu.VMEM((_B,), jnp.int32),
            pltpu.VMEM((_B, 128), jnp.int32),
            pltpu.VMEM((_LC, _D), jnp.float32),
            pltpu.SemaphoreType.DMA,
        ],
    )(*args)


def kernel(emb, q, c, a, qlen, clen, alen):
    emb = emb.astype(jnp.float32)
    ctok = c[:, :, 0].astype(jnp.int32)
    qtok = q[:, :, 0].astype(jnp.int32)
    clen = clen.astype(jnp.int32)
    qlen = qlen.astype(jnp.int32)
    alen = alen.astype(jnp.int32)
    a32 = a.astype(jnp.int32)

    pos = jnp.arange(_LC, dtype=jnp.int32)[None, :]
    cidx = jnp.where(pos < clen[:, None], ctok, ctok[:, :1])
    qidx = jnp.where(pos < qlen[:, None], qtok, qtok[:, :1])
    ccoef = (_LC - clen).astype(jnp.float32)
    qcoef = (_LC - qlen).astype(jnp.float32)

    idx_all = jnp.concatenate(
        [cidx, qidx, jnp.zeros((_NPAD - _NITEMS, _LC), jnp.int32)], axis=0)
    coef_all = jnp.concatenate(
        [ccoef, qcoef, jnp.zeros((_NPAD - _NITEMS,), jnp.float32)], axis=0)
    idx_flat = idx_all.reshape(-1)
    coef_b = coef_all[:, None] + jnp.zeros((_NPAD, _L), jnp.float32)
    a_flat = a32.reshape(-1)

    emb_pk = pl.pallas_call(
        _pack_tc_kernel,
        grid=(10,),
        in_specs=[pl.BlockSpec((_V // 10, _D), lambda i: (i, 0))],
        out_specs=pl.BlockSpec((_V // 10, _DP), lambda i: (i, 0)),
        out_shape=jax.ShapeDtypeStruct((_V, _DP), jnp.int32),
    )(emb)
    ssum, a_emb = _pool_call(emb_pk, emb, idx_flat, coef_b, a_flat)

    best = pl.pallas_call(
        _score_tc_kernel,
        out_shape=jax.ShapeDtypeStruct((_B,), jnp.int32),
    )(ssum)

    ctok_pad = jnp.pad(ctok, ((0, 0), (0, 128 - _LC)))
    c_emb = _gather_top_call(emb, ctok_pad, best)

    loss = pl.pallas_call(
        _loss_tc_kernel,
        grid=(_B,),
        in_specs=[
            pl.BlockSpec(memory_space=pltpu.SMEM),
            pl.BlockSpec((1, _LA, _D), lambda b: (b, 0, 0)),
            pl.BlockSpec((1, _LC, _D), lambda b: (b, 0, 0)),
        ],
        out_specs=pl.BlockSpec(memory_space=pltpu.SMEM),
        out_shape=jax.ShapeDtypeStruct((1, 1), jnp.float32),
    )(alen, a_emb, c_emb)

    return loss[0, 0]


# speedup vs baseline: 1.9825x; 1.0412x over previous
import functools

import jax
import jax.numpy as jnp
from jax import lax
from jax.experimental import pallas as pl
from jax.experimental.pallas import tpu as pltpu
from jax.experimental.pallas import tpu_sc as plsc

_NC, _NS, _L = 2, 16, 16
_NW = _NC * _NS

_V = 50000
_N = 4096
_B = 32
_LC = 32
_LA = 24
_D = 256
_NITEMS = _N + _B
_NPAD = 4352
_PER_W = _NPAD // _NW
_CH = 4
_NCHUNK = _PER_W // _CH
_NBUF = 2
_DP = _D // 2


def _pool_sc_kernel(embp_h, emb_h, idx_h, coef_h, a_h, out_h, aout_h,
                    idx_v, rows_v, coef_v, acc_v, aidx_v, arows_v,
                    sems, asem):
    w = lax.axis_index("s") * _NC + lax.axis_index("c")
    base = w * _PER_W

    pltpu.sync_copy(idx_h.at[pl.ds(base * _LC, _PER_W * _LC)], idx_v)
    pltpu.sync_copy(coef_h.at[pl.ds(base, _PER_W)], coef_v)
    pltpu.sync_copy(a_h.at[pl.ds(w * _LA, _LA)], aidx_v)
    pltpu.async_copy(emb_h.at[aidx_v], arows_v, asem)

    def _gather(ci, buf):
        return pltpu.make_async_copy(
            embp_h.at[idx_v.at[pl.ds(ci * _CH * _LC, _CH * _LC)]],
            rows_v.at[buf], sems.at[buf])

    def _fire(ci, buf):
        pltpu.async_copy(
            embp_h.at[idx_v.at[pl.ds(ci * _CH * _LC, _CH * _LC)]],
            rows_v.at[buf], sems.at[buf])

    for b0 in range(_NBUF):
        _fire(b0, b0)

    def outer(it, carry):
        for buf in range(_NBUF):
            ci = it * _NBUF + buf
            _gather(ci, buf).wait()

            def item_body(j, carry2):
                coefj = coef_v[ci * _CH + j]
                scale = 1.0 - coefj
                r0 = j * _LC

                def _row(l, sl):
                    x = rows_v[buf, r0 + l, sl]
                    lo = lax.bitcast_convert_type(x << 16, jnp.float32)
                    hi = lax.bitcast_convert_type(x & jnp.int32(-65536),
                                                  jnp.float32)
                    return lo, hi

                for g in range(_DP // _L):
                    sl = pl.ds(g * _L, _L)
                    e0, o0 = _row(0, sl)
                    ea, oa = [e0 * scale], [o0 * scale]
                    for k in range(1, 4):
                        ek, ok = _row(k, sl)
                        ea.append(ek)
                        oa.append(ok)
                    for lb in range(4, _LC, 4):
                        for k in range(4):
                            ek, ok = _row(lb + k, sl)
                            ea[k] = ea[k] + ek
                            oa[k] = oa[k] + ok
                    acc_e = (ea[0] + ea[1]) + (ea[2] + ea[3])
                    acc_o = (oa[0] + oa[1]) + (oa[2] + oa[3])
                    acc_v[buf * _CH + j, pl.ds(g * _L, _L)] = acc_e
                    acc_v[buf * _CH + j, pl.ds(_DP + g * _L, _L)] = acc_o
                return carry2

            lax.fori_loop(0, _CH, item_body, 0)

            @pl.when(ci + _NBUF < _NCHUNK)
            def _():
                _fire(ci + _NBUF, buf)

        pltpu.sync_copy(acc_v, out_h.at[pl.ds(base + it * _NBUF * _CH,
                                              _NBUF * _CH)])
        return carry

    lax.fori_loop(0, _NCHUNK // _NBUF, outer, 0)

    pltpu.make_async_copy(emb_h.at[aidx_v], arows_v, asem).wait()
    pltpu.sync_copy(arows_v, aout_h.at[w])


def _gather_top_sc_kernel(emb_h, ctok_h, best_h, cout_h,
                          bidx_v, ctoksel_v, cemb_v, sem):
    w = lax.axis_index("s") * _NC + lax.axis_index("c")
    pltpu.sync_copy(best_h, bidx_v)
    pltpu.async_copy(ctok_h.at[bidx_v], ctoksel_v, sem).wait()
    pltpu.async_copy(emb_h.at[ctoksel_v.at[w, pl.ds(0, _LC)]], cemb_v,
                     sem).wait()
    pltpu.sync_copy(cemb_v, cout_h.at[w])


def _pack_tc_kernel(e_ref, p_ref):
    bu = lax.bitcast_convert_type(e_ref[...], jnp.uint32)
    rb = (bu + jnp.uint32(0x7FFF) + ((bu >> 16) & jnp.uint32(1))) >> 16
    p_ref[...] = lax.bitcast_convert_type(
        rb[:, :_DP] | (rb[:, _DP:] << 16), jnp.int32)


def _score_tc_kernel(s_ref, best_ref):
    S = s_ref[...]
    cs = S[:_N, :]
    qs = S[_N:_N + _B, :]
    cn = cs * lax.rsqrt(jnp.sum(cs * cs, axis=1, keepdims=True) + 1e-30)
    scores = lax.dot_general(qs, cn, (((1,), (1,)), ((), ())),
                             preferred_element_type=jnp.float32)
    m = jnp.max(scores, axis=1, keepdims=True)
    ii = lax.broadcasted_iota(jnp.int32, scores.shape, 1)
    cand = jnp.where(scores >= m, ii, jnp.int32(2 ** 30))
    best_ref[...] = jnp.min(cand, axis=1)


_QB = 4


def _loss_tc_kernel(alen_ref, a_ref, c_ref, out_ref):
    g = pl.program_id(0)

    @pl.when(g == 0)
    def _():
        out_ref[0, 0] = 0.0

    for q in range(_QB):
        b = g * _QB + q
        A = a_ref[q]
        C = c_ref[q]
        an = A * lax.rsqrt(jnp.sum(A * A, axis=1, keepdims=True))
        cn = C * lax.rsqrt(jnp.sum(C * C, axis=1, keepdims=True))
        cos = lax.dot_general(an, cn, (((1,), (1,)), ((), ())),
                              preferred_element_type=jnp.float32)
        em = jnp.exp(-0.5 * (cos - 1.0) ** 2 / (0.001 ** 2))
        sm = em / (jnp.sum(em, axis=1, keepdims=True) + 1e-10)
        mm = jnp.sum(em * sm, axis=1, keepdims=True)
        al = alen_ref[b].astype(jnp.float32)
        mask = (lax.broadcasted_iota(jnp.int32, (_LA, 1), 0)
                < alen_ref[b]).astype(jnp.float32)
        tot = jnp.sum(mm * mask)
        out_ref[0, 0] += (1.0 - tot / al) / _B


def _sc_mesh():
    return plsc.VectorSubcoreMesh(core_axis_name="c", subcore_axis_name="s",
                                  num_cores=_NC, num_subcores=_NS)


def _pool_call(*args):
    return pl.kernel(
        _pool_sc_kernel,
        out_type=(jax.ShapeDtypeStruct((_NPAD, _D), jnp.float32),
                  jax.ShapeDtypeStruct((_B, _LA, _D), jnp.float32)),
        mesh=_sc_mesh(),
        scratch_types=[
            pltpu.VMEM((_PER_W * _LC,), jnp.int32),
            pltpu.VMEM((_NBUF, _CH * _LC, _DP), jnp.int32),
            pltpu.VMEM((_PER_W, _L), jnp.float32),
            pltpu.VMEM((_NBUF * _CH, _D), jnp.float32),
            pltpu.VMEM((_LA,), jnp.int32),
            pltpu.VMEM((_LA, _D), jnp.float32),
            pltpu.SemaphoreType.DMA((_NBUF,)),
            pltpu.SemaphoreType.DMA,
        ],
    )(*args)


def _gather_top_call(*args):
    return pl.kernel(
        _gather_top_sc_kernel,
        out_type=jax.ShapeDtypeStruct((_B, _LC, _D), jnp.float32),
        mesh=_sc_mesh(),
        scratch_types=[
            pltpu.VMEM((_B,), jnp.int32),
            pltpu.VMEM((_B, 128), jnp.int32),
            pltpu.VMEM((_LC, _D), jnp.float32),
            pltpu.SemaphoreType.DMA,
        ],
    )(*args)


def kernel(emb, q, c, a, qlen, clen, alen):
    emb = emb.astype(jnp.float32)
    ctok = c[:, :, 0].astype(jnp.int32)
    qtok = q[:, :, 0].astype(jnp.int32)
    clen = clen.astype(jnp.int32)
    qlen = qlen.astype(jnp.int32)
    alen = alen.astype(jnp.int32)
    a32 = a.astype(jnp.int32)

    pos = jnp.arange(_LC, dtype=jnp.int32)[None, :]
    cidx = jnp.where(pos < clen[:, None], ctok, ctok[:, :1])
    qidx = jnp.where(pos < qlen[:, None], qtok, qtok[:, :1])
    ccoef = (_LC - clen).astype(jnp.float32)
    qcoef = (_LC - qlen).astype(jnp.float32)

    idx_all = jnp.concatenate(
        [cidx, qidx, jnp.zeros((_NPAD - _NITEMS, _LC), jnp.int32)], axis=0)
    coef_all = jnp.concatenate(
        [ccoef, qcoef, jnp.zeros((_NPAD - _NITEMS,), jnp.float32)], axis=0)
    idx_flat = idx_all.reshape(-1)
    coef_b = coef_all[:, None] + jnp.zeros((_NPAD, _L), jnp.float32)
    a_flat = a32.reshape(-1)

    emb_pk = pl.pallas_call(
        _pack_tc_kernel,
        grid=(10,),
        in_specs=[pl.BlockSpec((_V // 10, _D), lambda i: (i, 0))],
        out_specs=pl.BlockSpec((_V // 10, _DP), lambda i: (i, 0)),
        out_shape=jax.ShapeDtypeStruct((_V, _DP), jnp.int32),
    )(emb)
    ssum, a_emb = _pool_call(emb_pk, emb, idx_flat, coef_b, a_flat)

    best = pl.pallas_call(
        _score_tc_kernel,
        out_shape=jax.ShapeDtypeStruct((_B,), jnp.int32),
    )(ssum)

    ctok_pad = jnp.pad(ctok, ((0, 0), (0, 128 - _LC)))
    c_emb = _gather_top_call(emb, ctok_pad, best)

    loss = pl.pallas_call(
        _loss_tc_kernel,
        grid=(_B // _QB,),
        in_specs=[
            pl.BlockSpec(memory_space=pltpu.SMEM),
            pl.BlockSpec((_QB, _LA, _D), lambda b: (b, 0, 0)),
            pl.BlockSpec((_QB, _LC, _D), lambda b: (b, 0, 0)),
        ],
        out_specs=pl.BlockSpec(memory_space=pltpu.SMEM),
        out_shape=jax.ShapeDtypeStruct((1, 1), jnp.float32),
    )(alen, a_emb, c_emb)

    return loss[0, 0]
